# baseline (speedup 1.0000x reference)
import functools

import jax
import jax.numpy as jnp
from jax import lax
from jax.experimental import pallas as pl
from jax.experimental.pallas import tpu as pltpu
from jax.experimental.pallas import tpu_sc as plsc

B = 32
H = 512
W = 512
NC = 2
NS = 16
L = 16
NW = NC * NS

SC_IMGS = 12
TC_IMGS = B - SC_IMGS
KROWS = SC_IMGS * H // NW

ROWS_PER_CHUNK = 32
NCHUNK = KROWS // ROWS_PER_CHUNK
GPW = W // L
GPR = 8

_mesh = plsc.VectorSubcoreMesh(core_axis_name="c", subcore_axis_name="s")


@functools.partial(
    pl.kernel,
    mesh=_mesh,
    out_type=jax.ShapeDtypeStruct((NW, L), jnp.int32),
    scratch_types=[
        pltpu.VMEM((ROWS_PER_CHUNK, W), jnp.float32),
        pltpu.VMEM((ROWS_PER_CHUNK, W), jnp.float32),
        pltpu.VMEM((ROWS_PER_CHUNK, W), jnp.float32),
        pltpu.VMEM((ROWS_PER_CHUNK, W), jnp.float32),
        pltpu.VMEM((L,), jnp.int32),
        pltpu.SemaphoreType.DMA,
        pltpu.SemaphoreType.DMA,
    ],
)
def _delta1_sc(pred_hbm, target_hbm, out_hbm, p0, p1, t0, t1, obuf, sem0, sem1):
    cid = lax.axis_index("c")
    sid = lax.axis_index("s")
    wid = sid * NC + cid
    g0 = wid * KROWS
    pbufs = (p0, p1)
    tbufs = (t0, t1)
    sems = (sem0, sem1)

    def start(c, slot):
        g = g0 + c * ROWS_PER_CHUNK
        img = lax.div(g, H)
        rows = pl.ds(lax.rem(g, H), ROWS_PER_CHUNK)
        pltpu.async_copy(pred_hbm.at[img, rows, :], pbufs[slot], sems[slot])
        pltpu.async_copy(target_hbm.at[img, rows, :], tbufs[slot], sems[slot])

    def wait(slot):
        img = lax.div(g0, H)
        rows = pl.ds(0, ROWS_PER_CHUNK)
        pltpu.make_async_copy(pred_hbm.at[img, rows, :], pbufs[slot], sems[slot]).wait()
        pltpu.make_async_copy(target_hbm.at[img, rows, :], tbufs[slot], sems[slot]).wait()

    start(0, 0)
    start(1, 1)
    zero = jnp.zeros((L,), jnp.int32)

    def outer(j, acc):
        for slot in range(2):
            c = 2 * j + slot
            wait(slot)
            pb = pbufs[slot]
            tb = tbufs[slot]

            def body(i, carry, pb=pb, tb=tb):
                accs = list(carry)
                row = lax.shift_right_logical(i, 2)
                col = pl.multiple_of((i & 3) * (GPR * L), GPR * L)
                for k in range(GPR):
                    p = pb[row, pl.ds(col + k * L, L)]
                    t = tb[row, pl.ds(col + k * L, L)]
                    corr = (p < 1.25 * t) & (t < 1.25 * p)
                    valid = t > 0.0
                    step = jnp.where(corr, 0x10001, jnp.where(valid, 0x10000, 0))
                    accs[k % 4] = accs[k % 4] + step
                return tuple(accs)

            acc = plsc.parallel_loop(0, ROWS_PER_CHUNK * (GPW // GPR), 1, carry=acc)(body)

            @pl.when(c + 2 < NCHUNK)
            def _():
                start(c + 2, slot)

        return acc

    acc = lax.fori_loop(0, NCHUNK // 2, outer, (zero, zero, zero, zero))
    a0, a1, a2, a3 = acc
    obuf[...] = (a0 + a1) + (a2 + a3)
    pltpu.sync_copy(obuf, out_hbm.at[wid])


RB = 512
TC_STEPS = TC_IMGS * (H // RB)


def _delta1_tc_body(p_ref, t_ref, o_ref):
    i = pl.program_id(0)

    @pl.when(i == 0)
    def _():
        o_ref[...] = jnp.zeros_like(o_ref)

    p = p_ref[0]
    t = t_ref[0]
    corr = (p < 1.25 * t) & (t < 1.25 * p)
    valid = t > 0.0
    cmat = jnp.sum(corr.astype(jnp.float32).reshape(RB // 8, 8, W), axis=0)
    vmat = jnp.sum(valid.astype(jnp.float32).reshape(RB // 8, 8, W), axis=0)
    o_ref[0] = o_ref[0] + cmat
    o_ref[1] = o_ref[1] + vmat


_delta1_tc = pl.pallas_call(
    _delta1_tc_body,
    grid=(TC_STEPS,),
    in_specs=[
        pl.BlockSpec((1, RB, W), lambda i: (i // (H // RB) + SC_IMGS, i % (H // RB), 0)),
        pl.BlockSpec((1, RB, W), lambda i: (i // (H // RB) + SC_IMGS, i % (H // RB), 0)),
    ],
    out_specs=pl.BlockSpec((2, 8, W), lambda i: (0, 0, 0)),
    out_shape=jax.ShapeDtypeStruct((2, 8, W), jnp.float32),
)


def kernel(pred, target):
    sc_partials = _delta1_sc(pred, target)
    tc_partials = _delta1_tc(pred, target)
    sum_c = jnp.sum(sc_partials & 0xFFFF).astype(jnp.float32) + jnp.sum(tc_partials[0])
    sum_v = jnp.sum(sc_partials >> 16).astype(jnp.float32) + jnp.sum(tc_partials[1])
    acc = sum_c / jnp.maximum(sum_v, 1.0)
    return jnp.where(sum_v < 10.0, jnp.float32(-1.0), acc)


# speedup vs baseline: 1.0438x; 1.0438x over previous
import functools

import jax
import jax.numpy as jnp
from jax import lax
from jax.experimental import pallas as pl
from jax.experimental.pallas import tpu as pltpu
from jax.experimental.pallas import tpu_sc as plsc

B = 32
H = 512
W = 512
NC = 2
NS = 16
L = 16
NW = NC * NS

SC_IMGS = 14
TC_IMGS = B - SC_IMGS
KROWS = SC_IMGS * H // NW

ROWS_PER_CHUNK = 32
NCHUNK = KROWS // ROWS_PER_CHUNK
GPW = W // L
GPR = 8

_mesh = plsc.VectorSubcoreMesh(core_axis_name="c", subcore_axis_name="s")


@functools.partial(
    pl.kernel,
    mesh=_mesh,
    out_type=jax.ShapeDtypeStruct((NW, L), jnp.int32),
    scratch_types=[
        pltpu.VMEM((ROWS_PER_CHUNK, W), jnp.float32),
        pltpu.VMEM((ROWS_PER_CHUNK, W), jnp.float32),
        pltpu.VMEM((ROWS_PER_CHUNK, W), jnp.float32),
        pltpu.VMEM((ROWS_PER_CHUNK, W), jnp.float32),
        pltpu.VMEM((L,), jnp.int32),
        pltpu.SemaphoreType.DMA,
        pltpu.SemaphoreType.DMA,
    ],
)
def _delta1_sc(pred_hbm, target_hbm, out_hbm, p0, p1, t0, t1, obuf, sem0, sem1):
    cid = lax.axis_index("c")
    sid = lax.axis_index("s")
    wid = sid * NC + cid
    g0 = wid * KROWS
    pbufs = (p0, p1)
    tbufs = (t0, t1)
    sems = (sem0, sem1)

    def start(c, slot):
        g = g0 + c * ROWS_PER_CHUNK
        img = lax.div(g, H)
        rows = pl.ds(lax.rem(g, H), ROWS_PER_CHUNK)
        pltpu.async_copy(pred_hbm.at[img, rows, :], pbufs[slot], sems[slot])
        pltpu.async_copy(target_hbm.at[img, rows, :], tbufs[slot], sems[slot])

    def wait(slot):
        img = lax.div(g0, H)
        rows = pl.ds(0, ROWS_PER_CHUNK)
        pltpu.make_async_copy(pred_hbm.at[img, rows, :], pbufs[slot], sems[slot]).wait()
        pltpu.make_async_copy(target_hbm.at[img, rows, :], tbufs[slot], sems[slot]).wait()

    start(0, 0)
    start(1, 1)
    zero = jnp.zeros((L,), jnp.int32)

    def outer(j, acc):
        for slot in range(2):
            c = 2 * j + slot
            wait(slot)
            pb = pbufs[slot]
            tb = tbufs[slot]

            def body(i, carry, pb=pb, tb=tb):
                accs = list(carry)
                row = lax.shift_right_logical(i, 2)
                col = pl.multiple_of((i & 3) * (GPR * L), GPR * L)
                for k in range(GPR):
                    p = pb[row, pl.ds(col + k * L, L)]
                    t = tb[row, pl.ds(col + k * L, L)]
                    corr = (p < 1.25 * t) & (t < 1.25 * p)
                    valid = t > 0.0
                    step = jnp.where(corr, 0x10001, jnp.where(valid, 0x10000, 0))
                    accs[k % 4] = accs[k % 4] + step
                return tuple(accs)

            acc = plsc.parallel_loop(0, ROWS_PER_CHUNK * (GPW // GPR), 1, carry=acc)(body)

            @pl.when(c + 2 < NCHUNK)
            def _():
                start(c + 2, slot)

        return acc

    acc = lax.fori_loop(0, NCHUNK // 2, outer, (zero, zero, zero, zero))

    if NCHUNK % 2:
        wait(0)

        def tail_body(i, carry):
            accs = list(carry)
            row = lax.shift_right_logical(i, 2)
            col = pl.multiple_of((i & 3) * (GPR * L), GPR * L)
            for k in range(GPR):
                p = p0[row, pl.ds(col + k * L, L)]
                t = t0[row, pl.ds(col + k * L, L)]
                corr = (p < 1.25 * t) & (t < 1.25 * p)
                valid = t > 0.0
                step = jnp.where(corr, 0x10001, jnp.where(valid, 0x10000, 0))
                accs[k % 4] = accs[k % 4] + step
            return tuple(accs)

        acc = plsc.parallel_loop(0, ROWS_PER_CHUNK * (GPW // GPR), 1, carry=acc)(tail_body)

    a0, a1, a2, a3 = acc
    obuf[...] = (a0 + a1) + (a2 + a3)
    pltpu.sync_copy(obuf, out_hbm.at[wid])


RB = 512
TC_STEPS = TC_IMGS * (H // RB)


def _delta1_tc_body(p_ref, t_ref, o_ref):
    i = pl.program_id(0)

    @pl.when(i == 0)
    def _():
        o_ref[...] = jnp.zeros_like(o_ref)

    p = p_ref[0]
    t = t_ref[0]
    corr = (p < 1.25 * t) & (t < 1.25 * p)
    valid = t > 0.0
    step = jnp.where(corr, 0x10001, jnp.where(valid, 0x10000, 0))
    o_ref[0] = o_ref[0] + jnp.sum(step.reshape(RB // 8, 8, W), axis=0)


_delta1_tc = pl.pallas_call(
    _delta1_tc_body,
    grid=(TC_STEPS,),
    in_specs=[
        pl.BlockSpec((1, RB, W), lambda i: (i // (H // RB) + SC_IMGS, i % (H // RB), 0)),
        pl.BlockSpec((1, RB, W), lambda i: (i // (H // RB) + SC_IMGS, i % (H // RB), 0)),
    ],
    out_specs=pl.BlockSpec((1, 8, W), lambda i: (0, 0, 0)),
    out_shape=jax.ShapeDtypeStruct((1, 8, W), jnp.int32),
)


def kernel(pred, target):
    sc_partials = _delta1_sc(pred, target)
    tc_partials = _delta1_tc(pred, target)
    sum_c = (jnp.sum(sc_partials & 0xFFFF) + jnp.sum(tc_partials & 0xFFFF)).astype(jnp.float32)
    sum_v = (jnp.sum(sc_partials >> 16) + jnp.sum(tc_partials >> 16)).astype(jnp.float32)
    acc = sum_c / jnp.maximum(sum_v, 1.0)
    return jnp.where(sum_v < 10.0, jnp.float32(-1.0), acc)
